# baseline (speedup 1.0000x reference)
import jax
import jax.numpy as jnp
from jax.experimental import pallas as pl
from jax.experimental.pallas import tpu as pltpu

_N = 8192
_BI = 256
_NI = _N // _BI
_K = 48


def _split3(x):
    h = jax.lax.optimization_barrier(x.astype(jnp.bfloat16))
    r = x - h.astype(jnp.float32)
    l = jax.lax.optimization_barrier(r.astype(jnp.bfloat16))
    r2 = r - l.astype(jnp.float32)
    l2 = jax.lax.optimization_barrier(r2.astype(jnp.bfloat16))
    return h, l, l2


def _lift(target, output):
    th, tl, tl2 = _split3(target)
    t2 = jnp.sum(target * target, axis=1, keepdims=True)
    t2h, t2l, t2l2 = _split3(t2)
    cols = []
    for c in range(3):
        for limb in (th, tl, tl2):
            one = limb[:, c : c + 1]
            cols.extend([one, one, one])
    cols.extend([t2h, t2l, t2l2])
    cols.append(jnp.ones((_N, 3), jnp.bfloat16))
    cols.append(jnp.zeros((_N, _K - 33), jnp.bfloat16))
    a = jnp.concatenate(cols, axis=1)

    ot = output.T
    oh, olo, olo2 = _split3(-2.0 * ot)
    o2 = jnp.sum(ot * ot, axis=0, keepdims=True)
    o2h, o2l, o2l2 = _split3(o2)
    rows = []
    for c in range(3):
        trio = jnp.concatenate(
            [oh[c : c + 1], olo[c : c + 1], olo2[c : c + 1]], axis=0
        )
        rows.extend([trio, trio, trio])
    rows.append(jnp.ones((3, _N), jnp.bfloat16))
    rows.append(jnp.concatenate([o2h, o2l, o2l2], axis=0))
    rows.append(jnp.zeros((_K - 33, _N), jnp.bfloat16))
    b = jnp.concatenate(rows, axis=0)
    return a, b


def _chamfer_body(a_ref, b_ref, out_ref, d2_ref, acc_ref):
    i = pl.program_id(0)
    d = jax.lax.dot_general(
        a_ref[...], b_ref[...], (((1,), (0,)), ((), ())),
        preferred_element_type=jnp.float32,
    )
    d1 = jnp.maximum(jnp.min(d, axis=1), 0.0)
    sq = jnp.sum(jnp.sqrt(d1))
    cmin = jnp.min(d, axis=0, keepdims=True)

    @pl.when(i == 0)
    def _():
        acc_ref[0, 0] = sq
        d2_ref[...] = cmin

    @pl.when(i > 0)
    def _():
        acc_ref[0, 0] = acc_ref[0, 0] + sq
        d2_ref[...] = jnp.minimum(d2_ref[...], cmin)

    @pl.when(i == _NI - 1)
    def _():
        d2 = jnp.maximum(d2_ref[...], 0.0)
        s2 = jnp.sum(jnp.sqrt(d2))
        loss = (acc_ref[0, 0] / _N + s2 / _N) * 5.0
        out_ref[...] = jnp.full((1, 1), loss, jnp.float32)


def kernel(target, output):
    a, b = _lift(target, output)
    out = pl.pallas_call(
        _chamfer_body,
        grid=(_NI,),
        in_specs=[
            pl.BlockSpec((_BI, _K), lambda i: (i, 0)),
            pl.BlockSpec((_K, _N), lambda i: (0, 0)),
        ],
        out_specs=pl.BlockSpec((1, 1), lambda i: (0, 0)),
        out_shape=jax.ShapeDtypeStruct((1, 1), jnp.float32),
        scratch_shapes=[
            pltpu.VMEM((1, _N), jnp.float32),
            pltpu.SMEM((1, 1), jnp.float32),
        ],
    )(a, b)
    return out[0, 0]


# speedup vs baseline: 1.1598x; 1.1598x over previous
import jax
import jax.numpy as jnp
from jax.experimental import pallas as pl
from jax.experimental.pallas import tpu as pltpu

_N = 8192
_BI = 256
_NI = _N // _BI
_K = 16


def _split3_f32(x):
    h = jax.lax.optimization_barrier(x.astype(jnp.bfloat16)).astype(jnp.float32)
    r = x - h
    l = jax.lax.optimization_barrier(r.astype(jnp.bfloat16)).astype(jnp.float32)
    r2 = r - l
    l2 = jax.lax.optimization_barrier(r2.astype(jnp.bfloat16)).astype(jnp.float32)
    return h, l, l2


def _lift(target, output):
    t2 = jnp.sum(target * target, axis=1, keepdims=True)
    t2h, t2l, t2l2 = _split3_f32(t2)
    a = jnp.concatenate(
        [
            target,
            t2h, t2l, t2l2,
            jnp.ones((_N, 3), jnp.float32),
            jnp.zeros((_N, _K - 9), jnp.float32),
        ],
        axis=1,
    )

    ot = output.T
    o2 = jnp.sum(ot * ot, axis=0, keepdims=True)
    o2h, o2l, o2l2 = _split3_f32(o2)
    b = jnp.concatenate(
        [
            -2.0 * ot,
            jnp.ones((3, _N), jnp.float32),
            o2h, o2l, o2l2,
            jnp.zeros((_K - 9, _N), jnp.float32),
        ],
        axis=0,
    )
    return a, b


def _chamfer_body(a_ref, b_ref, out_ref, d2_ref, acc_ref):
    i = pl.program_id(0)
    d = jax.lax.dot_general(
        a_ref[...], b_ref[...], (((1,), (0,)), ((), ())),
        preferred_element_type=jnp.float32,
    )
    d1 = jnp.maximum(jnp.min(d, axis=1), 0.0)
    sq = jnp.sum(jnp.sqrt(d1))
    cmin = jnp.min(d, axis=0, keepdims=True)

    @pl.when(i == 0)
    def _():
        acc_ref[0, 0] = sq
        d2_ref[...] = cmin

    @pl.when(i > 0)
    def _():
        acc_ref[0, 0] = acc_ref[0, 0] + sq
        d2_ref[...] = jnp.minimum(d2_ref[...], cmin)

    @pl.when(i == _NI - 1)
    def _():
        d2 = jnp.maximum(d2_ref[...], 0.0)
        s2 = jnp.sum(jnp.sqrt(d2))
        loss = (acc_ref[0, 0] / _N + s2 / _N) * 5.0
        out_ref[...] = jnp.full((1, 1), loss, jnp.float32)


def kernel(target, output):
    a, b = _lift(target, output)
    out = pl.pallas_call(
        _chamfer_body,
        grid=(_NI,),
        in_specs=[
            pl.BlockSpec((_BI, _K), lambda i: (i, 0)),
            pl.BlockSpec((_K, _N), lambda i: (0, 0)),
        ],
        out_specs=pl.BlockSpec((1, 1), lambda i: (0, 0)),
        out_shape=jax.ShapeDtypeStruct((1, 1), jnp.float32),
        scratch_shapes=[
            pltpu.VMEM((1, _N), jnp.float32),
            pltpu.SMEM((1, 1), jnp.float32),
        ],
    )(a, b)
    return out[0, 0]


# speedup vs baseline: 1.2968x; 1.1182x over previous
import jax
import jax.numpy as jnp
from jax.experimental import pallas as pl
from jax.experimental.pallas import tpu as pltpu

_N = 8192
_BI = 512
_NI = _N // _BI
_K = 16


def _split3_f32(x):
    h = jax.lax.optimization_barrier(x.astype(jnp.bfloat16)).astype(jnp.float32)
    r = x - h
    l = jax.lax.optimization_barrier(r.astype(jnp.bfloat16)).astype(jnp.float32)
    r2 = r - l
    l2 = jax.lax.optimization_barrier(r2.astype(jnp.bfloat16)).astype(jnp.float32)
    return h, l, l2


def _lift(target, output):
    t2 = jnp.sum(target * target, axis=1, keepdims=True)
    t2h, t2l, t2l2 = _split3_f32(t2)
    a = jnp.concatenate(
        [
            target,
            t2h, t2l, t2l2,
            jnp.ones((_N, 3), jnp.float32),
            jnp.zeros((_N, _K - 9), jnp.float32),
        ],
        axis=1,
    )

    ot = output.T
    o2 = jnp.sum(ot * ot, axis=0, keepdims=True)
    o2h, o2l, o2l2 = _split3_f32(o2)
    b = jnp.concatenate(
        [
            -2.0 * ot,
            jnp.ones((3, _N), jnp.float32),
            o2h, o2l, o2l2,
            jnp.zeros((_K - 9, _N), jnp.float32),
        ],
        axis=0,
    )
    return a, b


def _chamfer_body(a_ref, b_ref, out_ref, d2_ref, acc_ref):
    i = pl.program_id(0)
    d = jax.lax.dot_general(
        a_ref[...], b_ref[...], (((1,), (0,)), ((), ())),
        preferred_element_type=jnp.float32,
    )
    d1 = jnp.maximum(jnp.min(d, axis=1), 0.0)
    sq = jnp.sum(jnp.sqrt(d1))
    cmin = jnp.min(d, axis=0, keepdims=True)

    @pl.when(i == 0)
    def _():
        acc_ref[0, 0] = sq
        d2_ref[...] = cmin

    @pl.when(i > 0)
    def _():
        acc_ref[0, 0] = acc_ref[0, 0] + sq
        d2_ref[...] = jnp.minimum(d2_ref[...], cmin)

    @pl.when(i == _NI - 1)
    def _():
        d2 = jnp.maximum(d2_ref[...], 0.0)
        s2 = jnp.sum(jnp.sqrt(d2))
        loss = (acc_ref[0, 0] / _N + s2 / _N) * 5.0
        out_ref[...] = jnp.full((1, 1), loss, jnp.float32)


def kernel(target, output):
    a, b = _lift(target, output)
    out = pl.pallas_call(
        _chamfer_body,
        grid=(_NI,),
        in_specs=[
            pl.BlockSpec((_BI, _K), lambda i: (i, 0)),
            pl.BlockSpec((_K, _N), lambda i: (0, 0)),
        ],
        out_specs=pl.BlockSpec((1, 1), lambda i: (0, 0)),
        out_shape=jax.ShapeDtypeStruct((1, 1), jnp.float32),
        scratch_shapes=[
            pltpu.VMEM((1, _N), jnp.float32),
            pltpu.SMEM((1, 1), jnp.float32),
        ],
    )(a, b)
    return out[0, 0]


# speedup vs baseline: 1.3761x; 1.0612x over previous
import jax
import jax.numpy as jnp
from jax.experimental import pallas as pl
from jax.experimental.pallas import tpu as pltpu

_N = 8192
_BI = 1024
_NI = _N // _BI
_K = 16


def _split3_f32(x):
    h = jax.lax.optimization_barrier(x.astype(jnp.bfloat16)).astype(jnp.float32)
    r = x - h
    l = jax.lax.optimization_barrier(r.astype(jnp.bfloat16)).astype(jnp.float32)
    r2 = r - l
    l2 = jax.lax.optimization_barrier(r2.astype(jnp.bfloat16)).astype(jnp.float32)
    return h, l, l2


def _lift(target, output):
    t2 = jnp.sum(target * target, axis=1, keepdims=True)
    t2h, t2l, t2l2 = _split3_f32(t2)
    a = jnp.concatenate(
        [
            target,
            t2h, t2l, t2l2,
            jnp.ones((_N, 3), jnp.float32),
            jnp.zeros((_N, _K - 9), jnp.float32),
        ],
        axis=1,
    )

    ot = output.T
    o2 = jnp.sum(ot * ot, axis=0, keepdims=True)
    o2h, o2l, o2l2 = _split3_f32(o2)
    b = jnp.concatenate(
        [
            -2.0 * ot,
            jnp.ones((3, _N), jnp.float32),
            o2h, o2l, o2l2,
            jnp.zeros((_K - 9, _N), jnp.float32),
        ],
        axis=0,
    )
    return a, b


def _chamfer_body(a_ref, b_ref, out_ref, d2_ref, acc_ref):
    i = pl.program_id(0)
    d = jax.lax.dot_general(
        a_ref[...], b_ref[...], (((1,), (0,)), ((), ())),
        preferred_element_type=jnp.float32,
    )
    d1 = jnp.maximum(jnp.min(d, axis=1), 0.0)
    sq = jnp.sum(jnp.sqrt(d1))
    cmin = jnp.min(d, axis=0, keepdims=True)

    @pl.when(i == 0)
    def _():
        acc_ref[0, 0] = sq
        d2_ref[...] = cmin

    @pl.when(i > 0)
    def _():
        acc_ref[0, 0] = acc_ref[0, 0] + sq
        d2_ref[...] = jnp.minimum(d2_ref[...], cmin)

    @pl.when(i == _NI - 1)
    def _():
        d2 = jnp.maximum(d2_ref[...], 0.0)
        s2 = jnp.sum(jnp.sqrt(d2))
        loss = (acc_ref[0, 0] / _N + s2 / _N) * 5.0
        out_ref[...] = jnp.full((1, 1), loss, jnp.float32)


def kernel(target, output):
    a, b = _lift(target, output)
    out = pl.pallas_call(
        _chamfer_body,
        grid=(_NI,),
        in_specs=[
            pl.BlockSpec((_BI, _K), lambda i: (i, 0)),
            pl.BlockSpec((_K, _N), lambda i: (0, 0)),
        ],
        out_specs=pl.BlockSpec((1, 1), lambda i: (0, 0)),
        out_shape=jax.ShapeDtypeStruct((1, 1), jnp.float32),
        scratch_shapes=[
            pltpu.VMEM((1, _N), jnp.float32),
            pltpu.SMEM((1, 1), jnp.float32),
        ],
    )(a, b)
    return out[0, 0]


# speedup vs baseline: 1.4101x; 1.0247x over previous
import jax
import jax.numpy as jnp
from jax.experimental import pallas as pl
from jax.experimental.pallas import tpu as pltpu

_N = 8192
_BI = 2048
_NI = _N // _BI
_K = 16


def _split3_f32(x):
    h = jax.lax.optimization_barrier(x.astype(jnp.bfloat16)).astype(jnp.float32)
    r = x - h
    l = jax.lax.optimization_barrier(r.astype(jnp.bfloat16)).astype(jnp.float32)
    r2 = r - l
    l2 = jax.lax.optimization_barrier(r2.astype(jnp.bfloat16)).astype(jnp.float32)
    return h, l, l2


def _lift(target, output):
    t2 = jnp.sum(target * target, axis=1, keepdims=True)
    t2h, t2l, t2l2 = _split3_f32(t2)
    a = jnp.concatenate(
        [
            target,
            t2h, t2l, t2l2,
            jnp.ones((_N, 3), jnp.float32),
            jnp.zeros((_N, _K - 9), jnp.float32),
        ],
        axis=1,
    )

    ot = output.T
    o2 = jnp.sum(ot * ot, axis=0, keepdims=True)
    o2h, o2l, o2l2 = _split3_f32(o2)
    b = jnp.concatenate(
        [
            -2.0 * ot,
            jnp.ones((3, _N), jnp.float32),
            o2h, o2l, o2l2,
            jnp.zeros((_K - 9, _N), jnp.float32),
        ],
        axis=0,
    )
    return a, b


def _chamfer_body(a_ref, b_ref, out_ref, d2_ref, acc_ref):
    i = pl.program_id(0)
    d = jax.lax.dot_general(
        a_ref[...], b_ref[...], (((1,), (0,)), ((), ())),
        preferred_element_type=jnp.float32,
    )
    d1 = jnp.maximum(jnp.min(d, axis=1), 0.0)
    sq = jnp.sum(jnp.sqrt(d1))
    cmin = jnp.min(d, axis=0, keepdims=True)

    @pl.when(i == 0)
    def _():
        acc_ref[0, 0] = sq
        d2_ref[...] = cmin

    @pl.when(i > 0)
    def _():
        acc_ref[0, 0] = acc_ref[0, 0] + sq
        d2_ref[...] = jnp.minimum(d2_ref[...], cmin)

    @pl.when(i == _NI - 1)
    def _():
        d2 = jnp.maximum(d2_ref[...], 0.0)
        s2 = jnp.sum(jnp.sqrt(d2))
        loss = (acc_ref[0, 0] / _N + s2 / _N) * 5.0
        out_ref[...] = jnp.full((1, 1), loss, jnp.float32)


def kernel(target, output):
    a, b = _lift(target, output)
    out = pl.pallas_call(
        _chamfer_body,
        grid=(_NI,),
        in_specs=[
            pl.BlockSpec((_BI, _K), lambda i: (i, 0)),
            pl.BlockSpec((_K, _N), lambda i: (0, 0)),
        ],
        out_specs=pl.BlockSpec((1, 1), lambda i: (0, 0)),
        out_shape=jax.ShapeDtypeStruct((1, 1), jnp.float32),
        scratch_shapes=[
            pltpu.VMEM((1, _N), jnp.float32),
            pltpu.SMEM((1, 1), jnp.float32),
        ],
    )(a, b)
    return out[0, 0]
